# baseline (speedup 1.0000x reference)
import functools

import numpy as np
import jax
import jax.numpy as jnp
from jax.experimental import pallas as pl

RADIUS = 9.0
BI = 32


def _group_norm(x, num_groups, gamma, beta, eps=1e-05):
    n, c = x.shape
    g = x.T.reshape(num_groups, (c // num_groups) * n)
    mean = g.mean(axis=1, keepdims=True)
    var = g.var(axis=1, keepdims=True)
    g = (g - mean) * jax.lax.rsqrt(var + eps)
    return g.reshape(c, n).T * gamma[None, :] + beta[None, :]


def _pairwise_kernel(xi_ref, ni_ref, m_ref, ci_ref, rows_ref, w2t_ref, out_ref,
                     *, cuts, h_ch):
    pj = [rows_ref[d:d + 1, :] for d in range(3)]
    nj = [rows_ref[3 + d:4 + d, :] for d in range(3)]
    dx = pj[0] - xi_ref[:, 0:1]
    dy = pj[1] - xi_ref[:, 1:2]
    dz = pj[2] - xi_ref[:, 2:3]
    r2 = dx * dx + dy * dy + dz * dz
    dot = (ni_ref[:, 0:1] * nj[0] + ni_ref[:, 1:2] * nj[1]
           + ni_ref[:, 2:3] * nj[2])
    t = 2.0 - dot
    w = jnp.exp(-(r2 * (t * t)))
    x1 = []
    for c in range(cuts):
        z = (m_ref[:, 3 * c:3 * c + 1] * pj[0]
             + m_ref[:, 3 * c + 1:3 * c + 2] * pj[1]
             + m_ref[:, 3 * c + 2:3 * c + 3] * pj[2]
             + ci_ref[:, c:c + 1])
        x1.append(jnp.maximum(z, 0.0))
    outs = []
    for h in range(h_ch):
        z = w2t_ref[cuts:cuts + 1, h:h + 1]
        for c in range(cuts):
            z = z + w2t_ref[c:c + 1, h:h + 1] * x1[c]
        zr = jnp.maximum(z, 0.0)
        fh = rows_ref[6 + h:7 + h, :]
        outs.append(jnp.sum(w * zr * fh, axis=1, keepdims=True))
    out_ref[...] = jnp.concatenate(outs, axis=1)


def _pairwise_conv(pts_s, nuv, normals, f, p):
    n = pts_s.shape[0]
    cuts = p['conv_w1'].shape[0]
    h_ch = p['conv_w2'].shape[0]
    m = jnp.einsum('ck,ikd->icd', p['conv_w1'], nuv).reshape(n, 3 * cuts)
    ci = p['conv_b1'][None, :] - jnp.einsum('icd,id->ic',
                                            m.reshape(n, cuts, 3), pts_s)
    rows = jnp.concatenate(
        [pts_s.T, normals.T, f.T,
         jnp.zeros((2, n), jnp.float32)], axis=0)
    w2t = jnp.concatenate([p['conv_w2'].T, p['conv_b2'][None, :]], axis=0)
    w2t = jnp.pad(w2t, ((0, 16 - w2t.shape[0]), (0, 0)))

    kern = functools.partial(_pairwise_kernel, cuts=cuts, h_ch=h_ch)
    grid = (n // BI,)
    return pl.pallas_call(
        kern,
        grid=grid,
        in_specs=[
            pl.BlockSpec((BI, 3), lambda g: (g, 0)),
            pl.BlockSpec((BI, 3), lambda g: (g, 0)),
            pl.BlockSpec((BI, 3 * cuts), lambda g: (g, 0)),
            pl.BlockSpec((BI, cuts), lambda g: (g, 0)),
            pl.BlockSpec((6 + h_ch + 2, n), lambda g: (0, 0)),
            pl.BlockSpec((16, h_ch), lambda g: (0, 0)),
        ],
        out_specs=pl.BlockSpec((BI, h_ch), lambda g: (g, 0)),
        out_shape=jax.ShapeDtypeStruct((n, h_ch), jnp.float32),
    )(pts_s, normals, m, ci, rows, w2t)


def _leaky(x, slope=0.2):
    return jnp.where(x >= 0, x, slope * x)


def _conv_forward(pts_s, nuv, normals, feats, p):
    f = _leaky(feats @ p['w_in1'].T + p['b_in1'])
    f = _leaky(f @ p['w_in2'].T + p['b_in2'])
    f = _group_norm(f, 4, p['gn_in_w'], p['gn_in_b'])
    out = _pairwise_conv(pts_s, nuv, normals, f, p)
    o = _leaky(out @ p['w_out1'].T + p['b_out1'])
    o = _leaky(o @ p['w_out2'].T + p['b_out2'])
    return _group_norm(o, 4, p['gn_out_w'], p['gn_out_b'])


def kernel(features, points, nuv, params):
    pts_s = points / (np.sqrt(2.0) * RADIUS)
    normals = nuv[:, 0, :]
    x = features
    i = 0
    while ('layer%d' % i) in params:
        p = params['layer%d' % i]
        xi = _conv_forward(pts_s, nuv, normals, x, p)
        xi = jnp.maximum(xi @ p['ll_w1'].T + p['ll_b1'], 0.0) @ p['ll_w2'].T \
            + p['ll_b2']
        x = x @ p['lt_w'].T + p['lt_b']
        x = x + xi
        i += 1
    return x
